# Initial kernel scaffold; baseline (speedup 1.0000x reference)
import jax
import jax.numpy as jnp
from jax.experimental import pallas as pl


def kernel(x, edge_attr, atom_table, hybrid_table, bond_table):
    raise NotImplementedError("write your pallas kernel here")


# speedup vs baseline: 1.7890x; 1.7890x over previous
import functools

import jax
import jax.numpy as jnp
from jax import lax
from jax.experimental import pallas as pl
from jax.experimental.pallas import tpu as pltpu
from jax.experimental.pallas import tpu_sc as plsc

NN = 100000
NE = 3200000
CN = 400
CE = 2000
N_CHUNKS_NODE = NN // CN
N_CHUNKS_EDGE = NE // CE
NW = 32

_f32 = jnp.float32
_i32 = jnp.int32


def _sc_body(x_hbm, ea_hbm, at_hbm, ht_hbm, bt_hbm, nout_hbm, eout_hbm,
             at_v, ht_v, bt_v, xv, nov, eav, eov):
    c = lax.axis_index("c")
    s = lax.axis_index("s")
    wid = s * 2 + c

    pltpu.sync_copy(at_hbm, at_v)
    pltpu.sync_copy(ht_hbm, ht_v)
    pltpu.sync_copy(bt_hbm, bt_v)

    iota = lax.iota(_i32, 16)

    @pl.loop(wid, N_CHUNKS_NODE, step=NW)
    def _node_chunk(ci):
        base = ci * CN
        pltpu.sync_copy(x_hbm.at[pl.ds(base * 16, CN * 16)], xv)

        @pl.loop(0, CN // 16)
        def _grp(g):
            rows = g * 16 + iota
            x_off = rows * 16
            o_off = rows * 62
            sidx = plsc.load_gather(xv, [x_off]).astype(_i32) * 32
            hidx = plsc.load_gather(xv, [x_off + 1]).astype(_i32) * 16
            for cc in range(32):
                v = plsc.load_gather(at_v, [sidx + cc])
                plsc.store_scatter(nov, [o_off + cc], v)
            for cc in range(16):
                v = plsc.load_gather(ht_v, [hidx + cc])
                plsc.store_scatter(nov, [o_off + (32 + cc)], v)
            for cc in range(14):
                v = plsc.load_gather(xv, [x_off + (2 + cc)])
                plsc.store_scatter(nov, [o_off + (48 + cc)], v)

        pltpu.sync_copy(nov, nout_hbm.at[pl.ds(base * 62, CN * 62)])

    @pl.loop(wid, N_CHUNKS_EDGE, step=NW)
    def _edge_chunk(ci):
        base = ci * CE
        pltpu.sync_copy(ea_hbm.at[pl.ds(base * 2, CE * 2)], eav)

        @pl.loop(0, CE // 16)
        def _grp(g):
            rows = g * 16 + iota
            e_off = rows * 2
            o_off = rows * 17
            bidx = plsc.load_gather(eav, [e_off]).astype(_i32) * 16
            for cc in range(16):
                v = plsc.load_gather(bt_v, [bidx + cc])
                plsc.store_scatter(eov, [o_off + cc], v)
            d = plsc.load_gather(eav, [e_off + 1])
            plsc.store_scatter(eov, [o_off + 16], d)

        pltpu.sync_copy(eov, eout_hbm.at[pl.ds(base * 17, CE * 17)])


_OUT_TYPE = (
    jax.ShapeDtypeStruct((NN * 62,), _f32),
    jax.ShapeDtypeStruct((NE * 17,), _f32),
)

_SCRATCH = [
    pltpu.VMEM((39 * 32,), _f32),
    pltpu.VMEM((8 * 16,), _f32),
    pltpu.VMEM((6 * 16,), _f32),
    pltpu.VMEM((CN * 16,), _f32),
    pltpu.VMEM((CN * 62,), _f32),
    pltpu.VMEM((CE * 2,), _f32),
    pltpu.VMEM((CE * 17,), _f32),
]

_MESH = plsc.VectorSubcoreMesh(core_axis_name="c", subcore_axis_name="s")

_sc_call = functools.partial(
    pl.kernel,
    out_type=_OUT_TYPE,
    mesh=_MESH,
    scratch_types=_SCRATCH,
    compiler_params=pltpu.CompilerParams(needs_layout_passes=False),
)(_sc_body)


@jax.jit
def kernel(x, edge_attr, atom_table, hybrid_table, bond_table):
    node_flat, edge_flat = _sc_call(
        x.reshape(-1), edge_attr.reshape(-1), atom_table.reshape(-1),
        hybrid_table.reshape(-1), bond_table.reshape(-1))
    return node_flat.reshape(NN, 62), edge_flat.reshape(NE, 17)


# speedup vs baseline: 18.6661x; 10.4337x over previous
import functools

import jax
import jax.numpy as jnp
from jax import lax
from jax.experimental import pallas as pl
from jax.experimental.pallas import tpu as pltpu
from jax.experimental.pallas import tpu_sc as plsc

NN = 100000
NE = 3200000
CN = 512
CE = 2048
N_FULL_NODE = NN // CN
NODE_REM_BASE = N_FULL_NODE * CN
NODE_REM = 256
N_FULL_EDGE = NE // CE
EDGE_REM_BASE = N_FULL_EDGE * CE
EDGE_REM = 1024
NW = 32

_f32 = jnp.float32
_i32 = jnp.int32


def _cvt_idx(v, hi):
    i = v.astype(_i32)
    return jnp.minimum(jnp.maximum(i, 0), hi)


def _sc_body(xt_hbm, bond_hbm, dist_hbm, at_hbm, ht_hbm, bt_hbm,
             nout_hbm, eout_hbm, at_v, ht_v, bt_v, xv, nov, bv, dv, eov):
    c = lax.axis_index("c")
    s = lax.axis_index("s")
    wid = s * 2 + c

    pltpu.sync_copy(at_hbm, at_v)
    pltpu.sync_copy(ht_hbm, ht_v)
    pltpu.sync_copy(bt_hbm, bt_v)

    def node_chunk(base, width):
        pltpu.sync_copy(xt_hbm.at[pl.ds(0, 16), pl.ds(base, width)],
                        xv.at[pl.ds(0, 16), pl.ds(0, width)])

        @pl.loop(0, width // 16)
        def _grp(g):
            j = g * 16
            sidx = _cvt_idx(xv[0, pl.ds(j, 16)], 38)
            hidx = _cvt_idx(xv[1, pl.ds(j, 16)], 7)
            for r in range(32):
                nov[r, pl.ds(j, 16)] = plsc.load_gather(at_v, [sidx + 39 * r])
            for r in range(16):
                nov[32 + r, pl.ds(j, 16)] = plsc.load_gather(ht_v, [hidx + 8 * r])
            for r in range(14):
                nov[48 + r, pl.ds(j, 16)] = xv[2 + r, pl.ds(j, 16)]

        pltpu.sync_copy(nov.at[pl.ds(0, 64), pl.ds(0, width)],
                        nout_hbm.at[pl.ds(0, 64), pl.ds(base, width)])

    def edge_chunk(base, width):
        pltpu.sync_copy(bond_hbm.at[pl.ds(base, width)], bv.at[pl.ds(0, width)])
        pltpu.sync_copy(dist_hbm.at[pl.ds(base, width)], dv.at[pl.ds(0, width)])

        @pl.loop(0, width // 16)
        def _grp(g):
            j = g * 16
            bidx = _cvt_idx(bv[pl.ds(j, 16)], 5)
            for r in range(16):
                eov[r, pl.ds(j, 16)] = plsc.load_gather(bt_v, [bidx + 6 * r])
            eov[16, pl.ds(j, 16)] = dv[pl.ds(j, 16)]

        pltpu.sync_copy(eov.at[pl.ds(0, 24), pl.ds(0, width)],
                        eout_hbm.at[pl.ds(0, 24), pl.ds(base, width)])

    @pl.loop(wid, N_FULL_NODE, step=NW)
    def _node(ci):
        node_chunk(ci * CN, CN)

    @pl.when(wid == 1)
    def _node_rem():
        base = pl.multiple_of(wid * 0 + NODE_REM_BASE, 128)
        node_chunk(base, NODE_REM)

    @pl.loop(wid, N_FULL_EDGE, step=NW)
    def _edge(ci):
        edge_chunk(ci * CE, CE)

    @pl.when(wid == 3)
    def _edge_rem():
        edge_chunk(EDGE_REM_BASE, EDGE_REM)


_OUT_TYPE = (
    jax.ShapeDtypeStruct((62, NN), _f32),
    jax.ShapeDtypeStruct((17, NE), _f32),
)

_SCRATCH = [
    pltpu.VMEM((39 * 32,), _f32),
    pltpu.VMEM((8 * 16,), _f32),
    pltpu.VMEM((6 * 16,), _f32),
    pltpu.VMEM((16, CN), _f32),
    pltpu.VMEM((64, CN), _f32),
    pltpu.VMEM((CE,), _f32),
    pltpu.VMEM((CE,), _f32),
    pltpu.VMEM((24, CE), _f32),
]

_MESH = plsc.VectorSubcoreMesh(core_axis_name="c", subcore_axis_name="s")

_sc_call = functools.partial(
    pl.kernel,
    out_type=_OUT_TYPE,
    mesh=_MESH,
    scratch_types=_SCRATCH,
    compiler_params=pltpu.CompilerParams(needs_layout_passes=False),
)(_sc_body)


@jax.jit
def kernel(x, edge_attr, atom_table, hybrid_table, bond_table):
    node_t, edge_t = _sc_call(
        x.T, edge_attr[:, 0], edge_attr[:, 1], atom_table.T.reshape(-1),
        hybrid_table.T.reshape(-1), bond_table.T.reshape(-1))
    return node_t.T, edge_t.T


# speedup vs baseline: 20.9437x; 1.1220x over previous
import functools

import jax
import jax.numpy as jnp
from jax import lax
from jax.experimental import pallas as pl
from jax.experimental.pallas import tpu as pltpu
from jax.experimental.pallas import tpu_sc as plsc

NN = 100000
NE = 3200000
CN = 256
CE = 2048
N_FULL_NODE = NN // CN
NODE_REM_BASE = N_FULL_NODE * CN
NODE_REM = 256
N_FULL_EDGE = NE // CE
EDGE_REM_BASE = N_FULL_EDGE * CE
EDGE_REM = 1024
NW = 32
K_PAIRS = (N_FULL_EDGE // NW + 2) // 2 * 2

_f32 = jnp.float32
_i32 = jnp.int32


def _cvt_idx(v, hi):
    i = v.astype(_i32)
    return jnp.minimum(jnp.maximum(i, 0), hi)


def _sc_body(xt_hbm, bond_hbm, dist_hbm, at_hbm, ht_hbm, bt_hbm,
             nout_hbm, eout_hbm, at_v, ht_v, bt_v, xv, nov, bv, dv,
             eov0, eov1, sem0, sem1):
    c = lax.axis_index("c")
    s = lax.axis_index("s")
    wid = s * 2 + c

    pltpu.sync_copy(at_hbm, at_v)
    pltpu.sync_copy(ht_hbm, ht_v)
    pltpu.sync_copy(bt_hbm, bt_v)

    eovs = (eov0, eov1)
    sems = (sem0, sem1)

    def node_chunk(base, width):
        pltpu.sync_copy(xt_hbm.at[pl.ds(0, 16), pl.ds(base, width)],
                        xv.at[pl.ds(0, 16), pl.ds(0, width)])

        @pl.loop(0, width // 16)
        def _grp(g):
            j = g * 16
            sidx = _cvt_idx(xv[0, pl.ds(j, 16)], 38)
            hidx = _cvt_idx(xv[1, pl.ds(j, 16)], 7)
            for r in range(32):
                nov[r, pl.ds(j, 16)] = plsc.load_gather(at_v, [sidx + 39 * r])
            for r in range(16):
                nov[32 + r, pl.ds(j, 16)] = plsc.load_gather(ht_v, [hidx + 8 * r])
            for r in range(14):
                nov[48 + r, pl.ds(j, 16)] = xv[2 + r, pl.ds(j, 16)]

        pltpu.sync_copy(nov.at[pl.ds(0, 64), pl.ds(0, width)],
                        nout_hbm.at[pl.ds(0, 64), pl.ds(base, width)])

    def edge_compute(base, width, eov):
        pltpu.sync_copy(bond_hbm.at[pl.ds(base, width)], bv.at[pl.ds(0, width)])
        pltpu.sync_copy(dist_hbm.at[pl.ds(base, width)], dv.at[pl.ds(0, width)])

        @pl.loop(0, width // 16)
        def _grp(g):
            j = g * 16
            bidx = _cvt_idx(bv[pl.ds(j, 16)], 5)
            for r in range(16):
                eov[r, pl.ds(j, 16)] = plsc.load_gather(bt_v, [bidx + 6 * r])
            eov[16, pl.ds(j, 16)] = dv[pl.ds(j, 16)]

    def edge_out_slice(base):
        return eout_hbm.at[pl.ds(0, 24), pl.ds(base, CE)]

    @pl.loop(wid, N_FULL_NODE, step=NW)
    def _node(ci):
        node_chunk(ci * CN, CN)

    @pl.when(wid == 1)
    def _node_rem():
        base = pl.multiple_of(wid * 0 + NODE_REM_BASE, 128)
        node_chunk(base, NODE_REM)

    n_k = (N_FULL_EDGE - 1 - wid) // NW + 1

    @pl.loop(0, K_PAIRS, step=2)
    def _edge_pair(k2):
        for b in range(2):
            k = k2 + b
            ci = wid + k * NW

            @pl.when(k < n_k)
            def _do():
                @pl.when(k >= 2)
                def _drain():
                    pltpu.make_async_copy(
                        eovs[b], edge_out_slice(0), sems[b]).wait()

                edge_compute(ci * CE, CE, eovs[b])
                pltpu.async_copy(eovs[b], edge_out_slice(ci * CE), sems[b])

    for b in range(2):
        pltpu.make_async_copy(eovs[b], edge_out_slice(0), sems[b]).wait()

    @pl.when(wid == 3)
    def _edge_rem():
        edge_compute(EDGE_REM_BASE, EDGE_REM, eov0)
        pltpu.sync_copy(eov0.at[pl.ds(0, 24), pl.ds(0, EDGE_REM)],
                        eout_hbm.at[pl.ds(0, 24),
                                    pl.ds(EDGE_REM_BASE, EDGE_REM)])


_OUT_TYPE = (
    jax.ShapeDtypeStruct((62, NN), _f32),
    jax.ShapeDtypeStruct((17, NE), _f32),
)

_SCRATCH = [
    pltpu.VMEM((39 * 32,), _f32),
    pltpu.VMEM((8 * 16,), _f32),
    pltpu.VMEM((6 * 16,), _f32),
    pltpu.VMEM((16, CN), _f32),
    pltpu.VMEM((64, CN), _f32),
    pltpu.VMEM((CE,), _f32),
    pltpu.VMEM((CE,), _f32),
    pltpu.VMEM((24, CE), _f32),
    pltpu.VMEM((24, CE), _f32),
    pltpu.SemaphoreType.DMA,
    pltpu.SemaphoreType.DMA,
]

_MESH = plsc.VectorSubcoreMesh(core_axis_name="c", subcore_axis_name="s")

_sc_call = functools.partial(
    pl.kernel,
    out_type=_OUT_TYPE,
    mesh=_MESH,
    scratch_types=_SCRATCH,
    compiler_params=pltpu.CompilerParams(needs_layout_passes=False),
)(_sc_body)


@jax.jit
def kernel(x, edge_attr, atom_table, hybrid_table, bond_table):
    node_t, edge_t = _sc_call(
        x.T, edge_attr[:, 0], edge_attr[:, 1], atom_table.T.reshape(-1),
        hybrid_table.T.reshape(-1), bond_table.T.reshape(-1))
    return node_t.T, edge_t.T


# speedup vs baseline: 40.2695x; 1.9228x over previous
import functools

import jax
import jax.numpy as jnp
from jax import lax
from jax.experimental import pallas as pl
from jax.experimental.pallas import tpu as pltpu
from jax.experimental.pallas import tpu_sc as plsc

NN = 100000
NE = 3200000
CN = 256
CE = 2048
N_FULL_NODE = NN // CN
NODE_REM_BASE = N_FULL_NODE * CN
NODE_REM = 256
N_FULL_EDGE = NE // CE
EDGE_REM_BASE = N_FULL_EDGE * CE
EDGE_REM = 1024
NW = 32
K_PAIRS = (N_FULL_EDGE // NW + 2) // 2 * 2

_f32 = jnp.float32
_i32 = jnp.int32


def _cvt_idx(v, hi):
    i = v.astype(_i32)
    return jnp.minimum(jnp.maximum(i, 0), hi)


def _sc_body(xt_hbm, bond_hbm, dist_hbm, at_hbm, ht_hbm, bt_hbm,
             nout_hbm, eout_hbm, at_v, ht_v, bt_v, xv, nov, bv, dv,
             eov0, eov1, sem0, sem1):
    c = lax.axis_index("c")
    s = lax.axis_index("s")
    wid = s * 2 + c

    pltpu.sync_copy(at_hbm, at_v)
    pltpu.sync_copy(ht_hbm, ht_v)
    pltpu.sync_copy(bt_hbm, bt_v)

    eovs = (eov0, eov1)
    sems = (sem0, sem1)

    def node_chunk(base, width):
        pltpu.sync_copy(xt_hbm.at[pl.ds(0, 16), pl.ds(base, width)],
                        xv.at[pl.ds(0, 16), pl.ds(0, width)])

        @pl.loop(0, width // 16)
        def _grp(g):
            j = g * 16
            sidx = _cvt_idx(xv[0, pl.ds(j, 16)], 38)
            hidx = _cvt_idx(xv[1, pl.ds(j, 16)], 7)
            for r0 in (0, 16):
                vals = [plsc.load_gather(at_v, [sidx + 39 * (r0 + r)])
                        for r in range(16)]
                for r in range(16):
                    nov[r0 + r, pl.ds(j, 16)] = vals[r]
            vals = [plsc.load_gather(ht_v, [hidx + 8 * r]) for r in range(16)]
            for r in range(16):
                nov[32 + r, pl.ds(j, 16)] = vals[r]
            vals = [xv[2 + r, pl.ds(j, 16)] for r in range(14)]
            for r in range(14):
                nov[48 + r, pl.ds(j, 16)] = vals[r]

        pltpu.sync_copy(nov.at[pl.ds(0, 64), pl.ds(0, width)],
                        nout_hbm.at[pl.ds(0, 64), pl.ds(base, width)])

    def edge_compute(base, width, eov):
        pltpu.sync_copy(bond_hbm.at[pl.ds(base, width)], bv.at[pl.ds(0, width)])
        pltpu.sync_copy(dist_hbm.at[pl.ds(base, width)], dv.at[pl.ds(0, width)])

        @pl.loop(0, width // 16, unroll=2)
        def _grp(g):
            j = g * 16
            bidx = _cvt_idx(bv[pl.ds(j, 16)], 5)
            vals = [plsc.load_gather(bt_v, [bidx + 6 * r]) for r in range(16)]
            dval = dv[pl.ds(j, 16)]
            for r in range(16):
                eov[r, pl.ds(j, 16)] = vals[r]
            eov[16, pl.ds(j, 16)] = dval

    def edge_out_slice(base):
        return eout_hbm.at[pl.ds(0, 24), pl.ds(base, CE)]

    @pl.loop(wid, N_FULL_NODE, step=NW)
    def _node(ci):
        node_chunk(ci * CN, CN)

    @pl.when(wid == 1)
    def _node_rem():
        base = pl.multiple_of(wid * 0 + NODE_REM_BASE, 128)
        node_chunk(base, NODE_REM)

    n_k = (N_FULL_EDGE - 1 - wid) // NW + 1

    @pl.loop(0, K_PAIRS, step=2)
    def _edge_pair(k2):
        for b in range(2):
            k = k2 + b
            ci = wid + k * NW

            @pl.when(k < n_k)
            def _do():
                @pl.when(k >= 2)
                def _drain():
                    pltpu.make_async_copy(
                        eovs[b], edge_out_slice(0), sems[b]).wait()

                edge_compute(ci * CE, CE, eovs[b])
                pltpu.async_copy(eovs[b], edge_out_slice(ci * CE), sems[b])

    for b in range(2):
        pltpu.make_async_copy(eovs[b], edge_out_slice(0), sems[b]).wait()

    @pl.when(wid == 3)
    def _edge_rem():
        edge_compute(EDGE_REM_BASE, EDGE_REM, eov0)
        pltpu.sync_copy(eov0.at[pl.ds(0, 24), pl.ds(0, EDGE_REM)],
                        eout_hbm.at[pl.ds(0, 24),
                                    pl.ds(EDGE_REM_BASE, EDGE_REM)])


_OUT_TYPE = (
    jax.ShapeDtypeStruct((62, NN), _f32),
    jax.ShapeDtypeStruct((17, NE), _f32),
)

_SCRATCH = [
    pltpu.VMEM((39 * 32,), _f32),
    pltpu.VMEM((8 * 16,), _f32),
    pltpu.VMEM((6 * 16,), _f32),
    pltpu.VMEM((16, CN), _f32),
    pltpu.VMEM((64, CN), _f32),
    pltpu.VMEM((CE,), _f32),
    pltpu.VMEM((CE,), _f32),
    pltpu.VMEM((24, CE), _f32),
    pltpu.VMEM((24, CE), _f32),
    pltpu.SemaphoreType.DMA,
    pltpu.SemaphoreType.DMA,
]

_MESH = plsc.VectorSubcoreMesh(core_axis_name="c", subcore_axis_name="s")

_sc_call = functools.partial(
    pl.kernel,
    out_type=_OUT_TYPE,
    mesh=_MESH,
    scratch_types=_SCRATCH,
    compiler_params=pltpu.CompilerParams(needs_layout_passes=False),
)(_sc_body)


@jax.jit
def kernel(x, edge_attr, atom_table, hybrid_table, bond_table):
    node_t, edge_t = _sc_call(
        x.T, edge_attr[:, 0], edge_attr[:, 1], atom_table.T.reshape(-1),
        hybrid_table.T.reshape(-1), bond_table.T.reshape(-1))
    return node_t.T, edge_t.T
